# baseline (speedup 1.0000x reference)
import functools

import jax
import jax.numpy as jnp
from jax import lax
from jax.experimental import pallas as pl
from jax.experimental.pallas import tpu as pltpu
from jax.experimental.pallas import tpu_sc as plsc

B = 16384
D = 32
L = 16
NC = 2
NS = 16
NW = NC * NS
BPW = B // NW
CHUNK = 128
NCHUNK = BPW // CHUNK

_mesh = plsc.VectorSubcoreMesh(core_axis_name="c", subcore_axis_name="s")


@functools.partial(
    pl.kernel,
    out_type=jax.ShapeDtypeStruct((B,), jnp.float32),
    mesh=_mesh,
    compiler_params=pltpu.CompilerParams(
        needs_layout_passes=False, use_tc_tiling_on_sc=False),
    scratch_types=[
        pltpu.VMEM((NCHUNK, CHUNK), jnp.int32),
        pltpu.VMEM((NCHUNK, CHUNK), jnp.int32),
        pltpu.VMEM((BPW, D), jnp.float32),
        pltpu.VMEM((BPW, D), jnp.float32),
        pltpu.VMEM((BPW,), jnp.float32),
        pltpu.SemaphoreType.DMA,
    ],
)
def _cf_kernel(user_hbm, item_hbm, utab_hbm, itab_hbm, out_hbm,
               uidx_v, iidx_v, urows_v, irows_v, out_v, sem):
    wid = lax.axis_index("s") * NC + lax.axis_index("c")
    base = wid * BPW

    pltpu.sync_copy(user_hbm.at[pl.ds(wid * NCHUNK, NCHUNK)], uidx_v)
    pltpu.sync_copy(item_hbm.at[pl.ds(wid * NCHUNK, NCHUNK)], iidx_v)

    copies = []
    for j in range(NCHUNK):
        copies.append(pltpu.async_copy(
            utab_hbm.at[uidx_v.at[j]],
            urows_v.at[pl.ds(j * CHUNK, CHUNK)], sem))
        copies.append(pltpu.async_copy(
            itab_hbm.at[iidx_v.at[j]],
            irows_v.at[pl.ds(j * CHUNK, CHUNK)], sem))
    for cp in copies:
        cp.wait()

    lanes = lax.iota(jnp.int32, L)

    def block_body(blk, carry):
        rows = blk * L + lanes
        acc = jnp.zeros((L,), jnp.float32)
        for d in range(D):
            col = jnp.full((L,), d, jnp.int32)
            u = plsc.load_gather(urows_v, [rows, col])
            it = plsc.load_gather(irows_v, [rows, col])
            acc = acc + u * it
        out_v[pl.ds(blk * L, L)] = acc
        return carry

    lax.fori_loop(0, BPW // L, block_body, 0)

    pltpu.sync_copy(out_v, out_hbm.at[pl.ds(base, BPW)])


def kernel(user, item, user_table, item_table):
    user2 = user.reshape(NW * NCHUNK, CHUNK)
    item2 = item.reshape(NW * NCHUNK, CHUNK)
    return _cf_kernel(user2, item2, user_table, item_table)


# speedup vs baseline: 1.0534x; 1.0534x over previous
import functools

import jax
import jax.numpy as jnp
from jax import lax
from jax.experimental import pallas as pl
from jax.experimental.pallas import tpu as pltpu
from jax.experimental.pallas import tpu_sc as plsc

B = 16384
V = 1000000
D = 32
L = 16
NC = 2
NS = 16
NW = NC * NS
BPW = B // NW
CHUNK = 128
NCHUNK = BPW // CHUNK
PACK = 128 // D
RBLK = 512
NPACK = V // PACK
_GRID = (V + RBLK - 1) // RBLK


def _repack_body(t_ref, o_ref):
    t = t_ref[...]
    tt = lax.dot_general(t, jnp.eye(D, dtype=jnp.float32),
                         (((0,), (0,)), ((), ())),
                         preferred_element_type=jnp.float32)
    o_ref[...] = jnp.concatenate(
        [tt[128 * s:128 * (s + 1), :] for s in range(PACK)], axis=1)


_repack = pl.pallas_call(
    _repack_body,
    grid=(_GRID,),
    in_specs=[pl.BlockSpec((D, RBLK), lambda i: (0, i))],
    out_specs=pl.BlockSpec((RBLK // PACK, PACK * D), lambda i: (i, 0)),
    out_shape=jax.ShapeDtypeStruct((_GRID * (RBLK // PACK), PACK * D),
                                   jnp.float32),
)


_mesh = plsc.VectorSubcoreMesh(core_axis_name="c", subcore_axis_name="s")


@functools.partial(
    pl.kernel,
    out_type=jax.ShapeDtypeStruct((B,), jnp.float32),
    mesh=_mesh,
    compiler_params=pltpu.CompilerParams(
        needs_layout_passes=False, use_tc_tiling_on_sc=False),
    scratch_types=[
        pltpu.VMEM((NCHUNK, CHUNK), jnp.int32),
        pltpu.VMEM((NCHUNK, CHUNK), jnp.int32),
        pltpu.VMEM((NCHUNK, CHUNK), jnp.int32),
        pltpu.VMEM((NCHUNK, CHUNK), jnp.int32),
        pltpu.VMEM((CHUNK, PACK * D), jnp.float32),
        pltpu.VMEM((CHUNK, PACK * D), jnp.float32),
        pltpu.VMEM((BPW,), jnp.float32),
        pltpu.SemaphoreType.DMA,
    ],
)
def _cf_kernel(user_hbm, item_hbm, upack_hbm, ipack_hbm, out_hbm,
               uidx_v, iidx_v, uq_v, iq_v, ubuf_v, ibuf_v, out_v, sem):
    wid = lax.axis_index("s") * NC + lax.axis_index("c")
    base = wid * BPW

    pltpu.sync_copy(user_hbm.at[pl.ds(wid * NCHUNK, NCHUNK)], uidx_v)
    pltpu.sync_copy(item_hbm.at[pl.ds(wid * NCHUNK, NCHUNK)], iidx_v)

    def _q(r):
        return (lax.shift_left(lax.shift_right_logical(r, 9), 7)
                + (r & 127))

    def qbody(v, carry):
        j = v // (CHUNK // L)
        col = (v % (CHUNK // L)) * L
        uq_v[j, pl.ds(col, L)] = _q(uidx_v[j, pl.ds(col, L)])
        iq_v[j, pl.ds(col, L)] = _q(iidx_v[j, pl.ds(col, L)])
        return carry

    lax.fori_loop(0, NCHUNK * CHUNK // L, qbody, 0)

    lanes = lax.iota(jnp.int32, L)

    def chunk_body(j, carry):
        cu = pltpu.async_copy(upack_hbm.at[uq_v.at[j]], ubuf_v, sem)
        ci = pltpu.async_copy(ipack_hbm.at[iq_v.at[j]], ibuf_v, sem)
        cu.wait()
        ci.wait()
        for k in range(CHUNK // L):
            pvec = k * L + lanes
            ubase = lax.shift_left(
                lax.shift_right_logical(uidx_v[j, pl.ds(k * L, L)], 7) & 3, 5)
            ibase = lax.shift_left(
                lax.shift_right_logical(iidx_v[j, pl.ds(k * L, L)], 7) & 3, 5)
            acc = jnp.zeros((L,), jnp.float32)
            for d in range(D):
                u = plsc.load_gather(ubuf_v, [pvec, ubase + d])
                it = plsc.load_gather(ibuf_v, [pvec, ibase + d])
                acc = acc + u * it
            out_v[pl.ds(j * CHUNK + k * L, L)] = acc
        return carry

    lax.fori_loop(0, NCHUNK, chunk_body, 0)

    pltpu.sync_copy(out_v, out_hbm.at[pl.ds(base, BPW)])


def kernel(user, item, user_table, item_table):
    upack = _repack(user_table.T)
    ipack = _repack(item_table.T)
    user2 = user.reshape(NW * NCHUNK, CHUNK)
    item2 = item.reshape(NW * NCHUNK, CHUNK)
    return _cf_kernel(user2, item2, upack, ipack)
